# baseline (speedup 1.0000x reference)
import functools

import jax
import jax.numpy as jnp
from jax import lax
from jax.experimental import pallas as pl
from jax.experimental.pallas import tpu as pltpu

_PACK = 64
_ROW_TILE = 512


def _round_up(n, m):
    return (n + m - 1) // m * m


def _packed_linear_kernel(x_ref, w_ref, b_ref, o_ref):
    acc = lax.dot_general(
        x_ref[...],
        w_ref[...],
        dimension_numbers=(((1,), (0,)), ((), ())),
        preferred_element_type=jnp.float32,
    )
    o_ref[...] = (acc + b_ref[...]).astype(o_ref.dtype)


@jax.jit
def _forward(x, weight, bias2d):
    B, d_in = x.shape
    d_out = weight.shape[0]

    pack = _PACK
    row_tile = _ROW_TILE

    B_p = _round_up(B, pack * row_tile)
    if B_p != B:
        x = jnp.pad(x, ((0, B_p - B), (0, 0)))
    R = B_p // pack

    xp = x.reshape(R, pack * d_in)

    w_big = jnp.kron(jnp.eye(pack, dtype=weight.dtype), weight.T)
    b_big = jnp.tile(bias2d, (1, pack))

    out = pl.pallas_call(
        _packed_linear_kernel,
        out_shape=jax.ShapeDtypeStruct((R, pack * d_out), x.dtype),
        grid=(R // row_tile,),
        in_specs=[
            pl.BlockSpec((row_tile, pack * d_in), lambda i: (i, 0)),
            pl.BlockSpec((pack * d_in, pack * d_out), lambda i: (0, 0)),
            pl.BlockSpec((1, pack * d_out), lambda i: (0, 0)),
        ],
        out_specs=pl.BlockSpec((row_tile, pack * d_out), lambda i: (i, 0)),
        compiler_params=pltpu.CompilerParams(
            dimension_semantics=("parallel",),
        ),
        cost_estimate=pl.CostEstimate(
            flops=2 * R * (pack * d_in) * (pack * d_out),
            bytes_accessed=R * pack * (d_in + d_out) * 4
            + (pack * d_in * pack * d_out + pack * d_out) * 4,
            transcendentals=0,
        ),
    )(xp, w_big, b_big)

    out = out.reshape(B_p, d_out)
    if B_p != B:
        out = out[:B]
    return out


def kernel(x, weight, bias2d):
    return _forward(x, weight, bias2d)


# speedup vs baseline: 1.6921x; 1.6921x over previous
import functools

import jax
import jax.numpy as jnp
from jax import lax
from jax.experimental import pallas as pl
from jax.experimental.pallas import tpu as pltpu

_TILE = 4096
_DEPTH = 6
_NCORES = 2


def _round_up(n, m):
    return (n + m - 1) // m * m


def _ring_kernel(x_any, w_ref, b_ref, o_any, xb, ob, in_sems, out_sems,
                 *, n_tiles):
    core = pl.program_id(0)
    row0 = core * (n_tiles * _TILE)

    def in_copy(slot, t):
        return pltpu.make_async_copy(
            x_any.at[pl.ds(row0 + t * _TILE, _TILE), :],
            xb.at[slot],
            in_sems.at[slot],
        )

    def out_copy(slot, t):
        return pltpu.make_async_copy(
            ob.at[slot],
            o_any.at[pl.ds(row0 + t * _TILE, _TILE), :],
            out_sems.at[slot],
        )

    for s in range(min(_DEPTH, n_tiles)):
        in_copy(s, s).start()

    for t in range(n_tiles):
        slot = t % _DEPTH
        in_copy(slot, t).wait()
        if t >= _DEPTH:
            out_copy(slot, t - _DEPTH).wait()
        acc = lax.dot_general(
            xb[slot],
            w_ref[...],
            dimension_numbers=(((1,), (1,)), ((), ())),
            preferred_element_type=jnp.float32,
        )
        ob[slot] = acc + b_ref[...]
        out_copy(slot, t).start()
        if t + _DEPTH < n_tiles:
            in_copy(slot, t + _DEPTH).start()

    for t in range(max(0, n_tiles - _DEPTH), n_tiles):
        out_copy(t % _DEPTH, t).wait()


@jax.jit
def _forward(x, weight, bias2d):
    B, d_in = x.shape
    d_out = weight.shape[0]

    span = _NCORES * _TILE
    B_p = _round_up(B, span)
    if B_p != B:
        x = jnp.pad(x, ((0, B_p - B), (0, 0)))
    n_tiles = B_p // span

    kern = functools.partial(_ring_kernel, n_tiles=n_tiles)

    out = pl.pallas_call(
        kern,
        grid=(_NCORES,),
        in_specs=[
            pl.BlockSpec(memory_space=pl.ANY),
            pl.BlockSpec(memory_space=pltpu.MemorySpace.VMEM),
            pl.BlockSpec(memory_space=pltpu.MemorySpace.VMEM),
        ],
        out_specs=pl.BlockSpec(memory_space=pl.ANY),
        out_shape=jax.ShapeDtypeStruct((B_p, d_out), x.dtype),
        scratch_shapes=[
            pltpu.VMEM((_DEPTH, _TILE, d_in), jnp.float32),
            pltpu.VMEM((_DEPTH, _TILE, d_out), jnp.float32),
            pltpu.SemaphoreType.DMA((_DEPTH,)),
            pltpu.SemaphoreType.DMA((_DEPTH,)),
        ],
        compiler_params=pltpu.CompilerParams(
            dimension_semantics=("parallel",),
        ),
        cost_estimate=pl.CostEstimate(
            flops=2 * B_p * d_in * d_out,
            bytes_accessed=B_p * (d_in + d_out) * 4,
            transcendentals=0,
        ),
    )(x, weight, bias2d)

    if B_p != B:
        out = out[:B]
    return out


def kernel(x, weight, bias2d):
    return _forward(x, weight, bias2d)
